# baseline (speedup 1.0000x reference)
import functools

import jax
import jax.numpy as jnp
from jax import lax
from jax.experimental import pallas as pl
from jax.experimental.pallas import tpu as pltpu
from jax.experimental.pallas import tpu_sc as plsc

_N_TOKENS = 16384
_D_MODEL = 2048
_N_EXPERTS = 16
_TOP_K = 2
_BLK = 2048
_BLKS_A = 5
_N_A = _BLK * _BLKS_A
_LANES = 16
_N_TILES = 16


def _top2_softmax(logits):
    b = logits.shape[0]
    col = lax.broadcasted_iota(jnp.int32, (b, _N_EXPERTS), 1)
    m1 = jnp.max(logits, axis=-1, keepdims=True)
    i1 = jnp.min(jnp.where(logits == m1, col, _N_EXPERTS), axis=-1, keepdims=True)
    masked = jnp.where(col == i1, -jnp.inf, logits)
    m2 = jnp.max(masked, axis=-1, keepdims=True)
    i2 = jnp.min(jnp.where(masked == m2, col, _N_EXPERTS), axis=-1, keepdims=True)
    e2 = jnp.exp(m2 - m1)
    w1 = 1.0 / (1.0 + e2)
    return col, i1, i2, w1


def _router_body_a(x_ref, wt_ref, rw_ref, idx_ref, idx2_ref):
    logits = jnp.dot(x_ref[...], wt_ref[...], preferred_element_type=jnp.float32)
    _, i1, i2, w1 = _top2_softmax(logits)
    rw_ref[...] = jnp.concatenate([w1, 1.0 - w1], axis=1)
    idx = jnp.concatenate([i1, i2], axis=1)
    idx_ref[...] = idx
    idx2_ref[...] = idx


def _router_body_b(x_ref, wt_ref, rw_in, idx_in, rw_ref, idx_ref, cnt_ref, cnt_acc):
    del rw_in, idx_in
    step = pl.program_id(0)
    logits = jnp.dot(x_ref[...], wt_ref[...], preferred_element_type=jnp.float32)
    col, i1, i2, w1 = _top2_softmax(logits)
    rw_ref[...] = jnp.concatenate([w1, 1.0 - w1], axis=1)
    idx_ref[...] = jnp.concatenate([i1, i2], axis=1)

    onehot = (col == i1).astype(jnp.float32) + (col == i2).astype(jnp.float32)
    cnt = jnp.sum(onehot, axis=0, keepdims=True)

    @pl.when(step == 0)
    def _init():
        cnt_acc[...] = cnt

    @pl.when(step != 0)
    def _accum():
        cnt_acc[...] += cnt

    @pl.when(step == pl.num_programs(0) - 1)
    def _emit():
        cnt_ref[...] = cnt_acc[...]


def _tc_router_a(x, wt):
    n, d = x.shape
    return pl.pallas_call(
        _router_body_a,
        grid=(_BLKS_A,),
        in_specs=[
            pl.BlockSpec((_BLK, d), lambda i: (i, 0)),
            pl.BlockSpec((d, _N_EXPERTS), lambda i: (0, 0)),
        ],
        out_specs=[
            pl.BlockSpec((_BLK, _TOP_K), lambda i: (i, 0)),
            pl.BlockSpec((_BLK, _TOP_K), lambda i: (i, 0)),
            pl.BlockSpec((_BLK, _TOP_K), lambda i: (i, 0)),
        ],
        out_shape=[
            jax.ShapeDtypeStruct((n, _TOP_K), jnp.float32),
            jax.ShapeDtypeStruct((n, _TOP_K), jnp.int32),
            jax.ShapeDtypeStruct((_N_A, _TOP_K), jnp.int32),
        ],
        compiler_params=pltpu.CompilerParams(
            dimension_semantics=("arbitrary",),
        ),
    )(x, wt)


def _tc_router_b(x, wt, rw_buf, idx_buf):
    n, d = x.shape
    grid = n // _BLK - _BLKS_A
    return pl.pallas_call(
        _router_body_b,
        grid=(grid,),
        in_specs=[
            pl.BlockSpec((_BLK, d), lambda i: (i + _BLKS_A, 0)),
            pl.BlockSpec((d, _N_EXPERTS), lambda i: (0, 0)),
            pl.BlockSpec((_BLK, _TOP_K), lambda i: (i + _BLKS_A, 0)),
            pl.BlockSpec((_BLK, _TOP_K), lambda i: (i + _BLKS_A, 0)),
        ],
        out_specs=[
            pl.BlockSpec((_BLK, _TOP_K), lambda i: (i + _BLKS_A, 0)),
            pl.BlockSpec((_BLK, _TOP_K), lambda i: (i + _BLKS_A, 0)),
            pl.BlockSpec((1, _N_EXPERTS), lambda i: (0, 0)),
        ],
        out_shape=[
            jax.ShapeDtypeStruct((n, _TOP_K), jnp.float32),
            jax.ShapeDtypeStruct((n, _TOP_K), jnp.int32),
            jax.ShapeDtypeStruct((1, _N_EXPERTS), jnp.float32),
        ],
        scratch_shapes=[pltpu.VMEM((1, _N_EXPERTS), jnp.float32)],
        input_output_aliases={2: 0, 3: 1},
        compiler_params=pltpu.CompilerParams(
            dimension_semantics=("arbitrary",),
        ),
    )(x, wt, rw_buf, idx_buf)


def _sc_hist_body(idx_hbm, cnt_hbm, idx_v, cnt_v, all_v, shared):
    cid = lax.axis_index("c")
    sid = lax.axis_index("s")
    n_idx = _N_A * _TOP_K
    chunk = n_idx // _N_TILES

    @pl.when(cid == 0)
    def _hist():
        base = sid * chunk
        pltpu.sync_copy(idx_hbm.at[pl.ds(base, chunk)], idx_v)
        n_banks = 8
        for j in range(0, n_banks * _LANES, _LANES):
            cnt_v[pl.ds(j, _LANES)] = jnp.zeros((_LANES,), jnp.float32)
        ones = jnp.ones((_LANES,), jnp.float32)
        for g in range(chunk // _LANES):
            v = idx_v[pl.ds(g * _LANES, _LANES)]
            plsc.addupdate_scatter(cnt_v, (v + (g % n_banks) * _LANES,), ones)
        total = cnt_v[pl.ds(0, _LANES)]
        for b in range(1, n_banks):
            total = total + cnt_v[pl.ds(b * _LANES, _LANES)]
        cnt_v[pl.ds(0, _LANES)] = total
        pltpu.sync_copy(cnt_v.at[pl.ds(0, _LANES)], shared.at[sid])

    plsc.subcore_barrier()

    @pl.when(jnp.logical_and(cid == 0, sid == 0))
    def _reduce():
        pltpu.sync_copy(shared, all_v)
        total = all_v[0, :]
        for t in range(1, _N_TILES):
            total = total + all_v[t, :]
        all_v[0, :] = total
        pltpu.sync_copy(all_v.at[0], cnt_hbm.at[0])


def _sc_hist(idx_flat):
    mesh = plsc.VectorSubcoreMesh(core_axis_name="c", subcore_axis_name="s")
    chunk = _N_A * _TOP_K // _N_TILES
    f = pl.kernel(
        _sc_hist_body,
        out_type=jax.ShapeDtypeStruct((1, _LANES), jnp.float32),
        mesh=mesh,
        scratch_types=[
            pltpu.VMEM((chunk,), jnp.int32),
            pltpu.VMEM((128,), jnp.float32),
            pltpu.VMEM((_N_TILES, _LANES), jnp.float32),
            pltpu.VMEM_SHARED((_N_TILES, _LANES), jnp.float32),
        ],
        compiler_params=pltpu.CompilerParams(needs_layout_passes=False),
    )
    return f(idx_flat)


def _loss_body(ca_ref, cb_ref, loss_ref):
    c = ca_ref[...] + cb_ref[...]
    mean = jnp.sum(c) / _N_EXPERTS
    var = jnp.sum((c - mean) ** 2) / (_N_EXPERTS - 1)
    loss_ref[...] = (jnp.sqrt(var) / (mean + 1e-6) * 0.01).reshape(1, 1)


def _tc_loss(cnt_a, cnt_b):
    return pl.pallas_call(
        _loss_body,
        out_shape=jax.ShapeDtypeStruct((1, 1), jnp.float32),
    )(cnt_a, cnt_b)


@functools.partial(jax.jit, static_argnames=())
def kernel(x, W):
    wt = W.T
    rw_a, idx_a, idx_a2 = _tc_router_a(x, wt)
    cnt_a = _sc_hist(idx_a2.reshape(-1))
    rw, idx, cnt_b = _tc_router_b(x, wt, rw_a, idx_a)
    loss = _tc_loss(cnt_a, cnt_b)
    return rw, idx, loss.reshape(())


# speedup vs baseline: 1.1793x; 1.1793x over previous
import functools

import jax
import jax.numpy as jnp
from jax import lax
from jax.experimental import pallas as pl
from jax.experimental.pallas import tpu as pltpu
from jax.experimental.pallas import tpu_sc as plsc

_N_TOKENS = 16384
_D_MODEL = 2048
_N_EXPERTS = 16
_TOP_K = 2
_BLK = 2048
_LANES = 16
_N_TILES = 16


def _router_body(x_ref, wt_ref, rw_ref, idx_ref):
    logits = jnp.dot(x_ref[...], wt_ref[...], preferred_element_type=jnp.float32)
    b = logits.shape[0]
    col = lax.broadcasted_iota(jnp.int32, (b, _N_EXPERTS), 1)

    m1 = jnp.max(logits, axis=-1, keepdims=True)
    i1 = jnp.min(jnp.where(logits == m1, col, _N_EXPERTS), axis=-1, keepdims=True)
    masked = jnp.where(col == i1, -jnp.inf, logits)
    m2 = jnp.max(masked, axis=-1, keepdims=True)
    i2 = jnp.min(jnp.where(masked == m2, col, _N_EXPERTS), axis=-1, keepdims=True)

    e2 = jnp.exp(m2 - m1)
    w1 = 1.0 / (1.0 + e2)
    rw_ref[...] = jnp.concatenate([w1, 1.0 - w1], axis=1)
    idx_ref[...] = jnp.concatenate([i1, i2], axis=1)


def _tc_router(x, wt):
    n, d = x.shape
    grid = n // _BLK
    return pl.pallas_call(
        _router_body,
        grid=(grid,),
        in_specs=[
            pl.BlockSpec((_BLK, d), lambda i: (i, 0)),
            pl.BlockSpec((d, _N_EXPERTS), lambda i: (0, 0)),
        ],
        out_specs=[
            pl.BlockSpec((_BLK, _TOP_K), lambda i: (i, 0)),
            pl.BlockSpec((_BLK, _TOP_K), lambda i: (i, 0)),
        ],
        out_shape=[
            jax.ShapeDtypeStruct((n, _TOP_K), jnp.float32),
            jax.ShapeDtypeStruct((n, _TOP_K), jnp.int32),
        ],
        compiler_params=pltpu.CompilerParams(
            dimension_semantics=("arbitrary",),
        ),
    )(x, wt)


def _sc_loss_body(idx_hbm, loss_hbm, idx_v, cnt_v, all_v, shared, loss_v):
    cid = lax.axis_index("c")
    sid = lax.axis_index("s")
    n_idx = _N_TOKENS * _TOP_K
    chunk = n_idx // _N_TILES

    @pl.when(cid == 0)
    def _hist():
        base = sid * chunk
        pltpu.sync_copy(idx_hbm.at[pl.ds(base, chunk)], idx_v)
        n_banks = 8
        for j in range(0, n_banks * _LANES, _LANES):
            cnt_v[pl.ds(j, _LANES)] = jnp.zeros((_LANES,), jnp.float32)
        ones = jnp.ones((_LANES,), jnp.float32)
        for g in range(chunk // _LANES):
            v = idx_v[pl.ds(g * _LANES, _LANES)]
            plsc.addupdate_scatter(cnt_v, (v + (g % n_banks) * _LANES,), ones)
        total = cnt_v[pl.ds(0, _LANES)]
        for b in range(1, n_banks):
            total = total + cnt_v[pl.ds(b * _LANES, _LANES)]
        cnt_v[pl.ds(0, _LANES)] = total
        pltpu.sync_copy(cnt_v.at[pl.ds(0, _LANES)], shared.at[sid])

    plsc.subcore_barrier()

    @pl.when(jnp.logical_and(cid == 0, sid == 0))
    def _loss():
        pltpu.sync_copy(shared, all_v)
        total = all_v[0, :]
        for t in range(1, _N_TILES):
            total = total + all_v[t, :]
        zero = jnp.zeros((_LANES,), jnp.float32)
        meanv = (zero + jnp.sum(total, axis=0)) * (1.0 / _N_EXPERTS)
        d = total - meanv
        varv = (zero + jnp.sum(d * d, axis=0)) * (1.0 / (_N_EXPERTS - 1))
        i = plsc.bitcast(varv, jnp.int32)
        y = plsc.bitcast(0x5F3759DF - (i >> 1), jnp.float32)
        for _ in range(3):
            y = y * (1.5 - 0.5 * varv * y * y)
        std = jnp.where(varv > 0, varv * y, zero)
        denom = meanv + 1e-6
        r = plsc.bitcast(0x7EF311C3 - plsc.bitcast(denom, jnp.int32), jnp.float32)
        for _ in range(3):
            r = r * (2.0 - denom * r)
        loss_v[...] = std * r * 0.01
        pltpu.sync_copy(loss_v, loss_hbm)


def _sc_loss(idx_flat):
    mesh = plsc.VectorSubcoreMesh(core_axis_name="c", subcore_axis_name="s", num_cores=1)
    chunk = _N_TOKENS * _TOP_K // _N_TILES
    f = pl.kernel(
        _sc_loss_body,
        out_type=jax.ShapeDtypeStruct((_LANES,), jnp.float32),
        mesh=mesh,
        scratch_types=[
            pltpu.VMEM((chunk,), jnp.int32),
            pltpu.VMEM((128,), jnp.float32),
            pltpu.VMEM((_N_TILES, _LANES), jnp.float32),
            pltpu.VMEM_SHARED((_N_TILES, _LANES), jnp.float32),
            pltpu.VMEM((_LANES,), jnp.float32),
        ],
        compiler_params=pltpu.CompilerParams(needs_layout_passes=False),
    )
    return f(idx_flat)


@functools.partial(jax.jit, static_argnames=())
def kernel(x, W):
    rw, idx = _tc_router(x, W.T)
    loss = _sc_loss(idx.reshape(-1))
    return rw, idx, loss[0].reshape(())
